# baseline (speedup 1.0000x reference)
import functools

import jax
import jax.numpy as jnp
from jax import lax
from jax.experimental import pallas as pl
from jax.experimental.pallas import tpu as pltpu
from jax.experimental.pallas import tpu_sc as plsc

VOCAB = 49408
N_WORDS = 77
D = 768
B = 1024

NW = 32
TOTAL = B * N_WORDS
PER_W = TOTAL // NW
CHUNK = 32
N_CHUNKS = PER_W // CHUNK
LANES = 16
D_SLICES = D // LANES


def _sc_embed(tok_ids, pos_ids, tok_emb, pos_emb):
    mesh = plsc.VectorSubcoreMesh(core_axis_name="c", subcore_axis_name="s")

    @functools.partial(
        pl.kernel,
        mesh=mesh,
        out_type=jax.ShapeDtypeStruct((TOTAL, D), jnp.float32),
        scratch_types=[
            pltpu.VMEM((N_CHUNKS, CHUNK), jnp.int32),
            pltpu.VMEM((N_CHUNKS, CHUNK), jnp.int32),
            pltpu.VMEM((N_WORDS, D), jnp.float32),
            pltpu.VMEM((CHUNK, D), jnp.float32),
            pltpu.VMEM((CHUNK, D), jnp.float32),
            pltpu.SemaphoreType.DMA,
            pltpu.SemaphoreType.DMA,
            pltpu.SemaphoreType.DMA,
            pltpu.SemaphoreType.DMA,
        ],
    )
    def k(tok_ids_hbm, pos_ids_hbm, tok_emb_hbm, pos_emb_hbm, out_hbm,
          tok_idx_v, pos_idx_v, pos_tab_v, rows0, rows1,
          gsem0, gsem1, wsem0, wsem1):
        wid = lax.axis_index("s") * 2 + lax.axis_index("c")
        base = wid * PER_W
        pltpu.sync_copy(tok_ids_hbm.at[wid], tok_idx_v)
        pltpu.sync_copy(pos_ids_hbm.at[wid], pos_idx_v)
        pltpu.sync_copy(pos_emb_hbm, pos_tab_v)

        pltpu.async_copy(tok_emb_hbm.at[tok_idx_v.at[0]], rows0, gsem0)

        def make_step(cur, gsem_c, wsem_c, nxt, gsem_n, wsem_n):
            def step(i):
                @pl.when(i + 1 < N_CHUNKS)
                def _issue_next():
                    @pl.when(i >= 1)
                    def _drain_write():
                        pltpu.make_async_copy(
                            nxt, out_hbm.at[pl.ds(0, CHUNK)], wsem_n).wait()
                    pltpu.async_copy(
                        tok_emb_hbm.at[tok_idx_v.at[i + 1]], nxt, gsem_n)

                pltpu.make_async_copy(
                    tok_emb_hbm.at[tok_idx_v.at[i]], cur, gsem_c).wait()

                def group_body(g, carry):
                    pid_vec = pos_idx_v[i, pl.ds(g * LANES, LANES)]

                    def add_row(r, pid):
                        def d_body(j, carry2):
                            sl = pl.ds(j * LANES, LANES)
                            cur[r, sl] = cur[r, sl] + pos_tab_v[pid, sl]
                            return carry2
                        lax.fori_loop(0, D_SLICES, d_body, 0, unroll=8)

                    for l in range(LANES):
                        add_row(g * LANES + l, pid_vec[l])
                    return carry

                lax.fori_loop(0, CHUNK // LANES, group_body, 0)
                pltpu.async_copy(
                    cur, out_hbm.at[pl.ds(base + i * CHUNK, CHUNK)], wsem_c)
            return step

        step_even = make_step(rows0, gsem0, wsem0, rows1, gsem1, wsem1)
        step_odd = make_step(rows1, gsem1, wsem1, rows0, gsem0, wsem0)

        def chunk_body(i, carry):
            @pl.when(i % 2 == 0)
            def _even():
                step_even(i)

            @pl.when(i % 2 == 1)
            def _odd():
                step_odd(i)
            return carry

        lax.fori_loop(0, N_CHUNKS, chunk_body, 0)

        pltpu.make_async_copy(rows0, out_hbm.at[pl.ds(0, CHUNK)], wsem0).wait()
        pltpu.make_async_copy(rows1, out_hbm.at[pl.ds(0, CHUNK)], wsem1).wait()

    return k(tok_ids, pos_ids, tok_emb, pos_emb)


def kernel(input_ids, position_ids, token_embedding, position_embedding):
    tok_ids = input_ids.reshape(NW, N_CHUNKS, CHUNK).astype(jnp.int32)
    pos_ids = position_ids.reshape(NW, N_CHUNKS, CHUNK).astype(jnp.int32)
    out = _sc_embed(tok_ids, pos_ids, token_embedding, position_embedding)
    return out.reshape(B, N_WORDS, D)


# speedup vs baseline: 1.3959x; 1.3959x over previous
import functools

import jax
import jax.numpy as jnp
from jax import lax
from jax.experimental import pallas as pl
from jax.experimental.pallas import tpu as pltpu
from jax.experimental.pallas import tpu_sc as plsc

VOCAB = 49408
N_WORDS = 77
D = 768
B = 1024

NW = 32
TOTAL = B * N_WORDS
PER_W = TOTAL // NW
CHUNK = 32
N_CHUNKS = PER_W // CHUNK
LANES = 16
D_SLICES = D // LANES


def _sc_embed(tok_ids, pos_ids, tok_emb, pos_emb):
    mesh = plsc.VectorSubcoreMesh(core_axis_name="c", subcore_axis_name="s")

    @functools.partial(
        pl.kernel,
        mesh=mesh,
        out_type=jax.ShapeDtypeStruct((TOTAL, D), jnp.float32),
        scratch_types=[
            pltpu.VMEM((N_CHUNKS, CHUNK), jnp.int32),
            pltpu.VMEM((N_CHUNKS, CHUNK), jnp.int32),
            pltpu.VMEM((CHUNK, D), jnp.float32),
            pltpu.VMEM((CHUNK, D), jnp.float32),
            pltpu.VMEM((CHUNK, D), jnp.float32),
            pltpu.VMEM((CHUNK, D), jnp.float32),
            pltpu.SemaphoreType.DMA,
            pltpu.SemaphoreType.DMA,
            pltpu.SemaphoreType.DMA,
            pltpu.SemaphoreType.DMA,
        ],
    )
    def k(tok_ids_hbm, pos_ids_hbm, tok_emb_hbm, pos_emb_hbm, out_hbm,
          tok_idx_v, pos_idx_v, trows0, trows1, prows0, prows1,
          gsem0, gsem1, wsem0, wsem1):
        wid = lax.axis_index("s") * 2 + lax.axis_index("c")
        base = wid * PER_W
        pltpu.sync_copy(tok_ids_hbm.at[wid], tok_idx_v)
        pltpu.sync_copy(pos_ids_hbm.at[wid], pos_idx_v)

        pltpu.async_copy(tok_emb_hbm.at[tok_idx_v.at[0]], trows0, gsem0)
        pltpu.async_copy(pos_emb_hbm.at[pos_idx_v.at[0]], prows0, gsem0)

        def make_step(tcur, pcur, gsem_c, wsem_c, tnxt, pnxt, gsem_n, wsem_n):
            def step(i):
                @pl.when(i + 1 < N_CHUNKS)
                def _issue_next():
                    @pl.when(i >= 1)
                    def _drain_write():
                        pltpu.make_async_copy(
                            tnxt, out_hbm.at[pl.ds(0, CHUNK)], wsem_n).wait()
                    pltpu.async_copy(
                        tok_emb_hbm.at[tok_idx_v.at[i + 1]], tnxt, gsem_n)
                    pltpu.async_copy(
                        pos_emb_hbm.at[pos_idx_v.at[i + 1]], pnxt, gsem_n)

                pltpu.make_async_copy(
                    tok_emb_hbm.at[tok_idx_v.at[i]], tcur, gsem_c).wait()
                pltpu.make_async_copy(
                    pos_emb_hbm.at[pos_idx_v.at[i]], pcur, gsem_c).wait()

                def row_body(r, carry):
                    for j in range(D_SLICES):
                        sl = pl.ds(j * LANES, LANES)
                        tcur[r, sl] = tcur[r, sl] + pcur[r, sl]
                    return carry

                lax.fori_loop(0, CHUNK, row_body, 0)
                pltpu.async_copy(
                    tcur, out_hbm.at[pl.ds(base + i * CHUNK, CHUNK)], wsem_c)
            return step

        step_even = make_step(trows0, prows0, gsem0, wsem0,
                              trows1, prows1, gsem1, wsem1)
        step_odd = make_step(trows1, prows1, gsem1, wsem1,
                             trows0, prows0, gsem0, wsem0)

        def chunk_body(i, carry):
            @pl.when(i % 2 == 0)
            def _even():
                step_even(i)

            @pl.when(i % 2 == 1)
            def _odd():
                step_odd(i)
            return carry

        lax.fori_loop(0, N_CHUNKS, chunk_body, 0)

        pltpu.make_async_copy(trows0, out_hbm.at[pl.ds(0, CHUNK)], wsem0).wait()
        pltpu.make_async_copy(trows1, out_hbm.at[pl.ds(0, CHUNK)], wsem1).wait()

    return k(tok_ids, pos_ids, tok_emb, pos_emb)


def kernel(input_ids, position_ids, token_embedding, position_embedding):
    tok_ids = input_ids.reshape(NW, N_CHUNKS, CHUNK).astype(jnp.int32)
    pos_ids = position_ids.reshape(NW, N_CHUNKS, CHUNK).astype(jnp.int32)
    out = _sc_embed(tok_ids, pos_ids, token_embedding, position_embedding)
    return out.reshape(B, N_WORDS, D)
